# baseline (speedup 1.0000x reference)
import jax
import jax.numpy as jnp
from jax import lax
from jax.experimental import pallas as pl
from jax.experimental.pallas import tpu as pltpu
from jax.experimental.pallas import tpu_sc as plsc

H = 128
NCLS = 128
BLK = 80
NW = 32
NITER = 125
NBUF = 5
WIN = 256


def _t12_body(emb_ref, w_ref, b_ref, out_ref):
    emb = emb_ref[...]
    t1 = jnp.dot(emb, w_ref[:H, :], preferred_element_type=jnp.float32)
    t2 = jnp.dot(emb, w_ref[H:, :], preferred_element_type=jnp.float32)
    t1 = t1 + b_ref[...]
    out_ref[...] = t1[:, None, :] + t2[None, :, :]


def _build_t12(emb_table, W, b):
    out = pl.pallas_call(
        _t12_body,
        out_shape=jax.ShapeDtypeStruct((NCLS, NCLS, H), jnp.float32),
    )(emb_table, W, b.reshape(1, H))
    return out.reshape(NCLS * NCLS, H)


def _edge_body(z_hbm, ei_hbm, t12_hbm, out_hbm, *scr):
    z_v = scr[0]
    sd_v = scr[1:1 + NBUF]
    cc_v = scr[1 + NBUF:1 + 2 * NBUF]
    rows_v = scr[1 + 2 * NBUF:1 + 3 * NBUF]
    isem = scr[1 + 3 * NBUF:1 + 4 * NBUF]
    gsem = scr[1 + 4 * NBUF:1 + 5 * NBUF]
    ssem = scr[1 + 5 * NBUF:1 + 6 * NBUF]

    cid = lax.axis_index("c")
    sid = lax.axis_index("s")
    wid = sid * 2 + cid

    def blkno(i):
        return wid + i * NW

    pltpu.sync_copy(z_hbm, z_v)

    def idx_start(i, b):
        st = blkno(i) * BLK
        st_al = (st // 128) * 128
        pltpu.async_copy(ei_hbm.at[:, pl.ds(st_al, WIN)], sd_v[b], isem[b])

    def gather_start(i, b, prefetch_idx=True, wait_store=True):
        pltpu.make_async_copy(ei_hbm.at[:, pl.ds(0, WIN)], sd_v[b],
                              isem[b]).wait()
        st = blkno(i) * BLK
        off = st - (st // 128) * 128
        for j in range(BLK // 16):
            s = plsc.load_gather(z_v, [sd_v[b][0, pl.ds(off + j * 16, 16)]])
            d = plsc.load_gather(z_v, [sd_v[b][1, pl.ds(off + j * 16, 16)]])
            cc_v[b][pl.ds(j * 16, 16)] = s * NCLS + d
        if prefetch_idx:
            idx_start(i + NBUF, b)
        if wait_store:
            pltpu.make_async_copy(out_hbm.at[pl.ds(0, BLK)], rows_v[b],
                                  ssem[b]).wait()
        pltpu.async_copy(t12_hbm.at[cc_v[b]], rows_v[b], gsem[b])

    def finish(i, b):
        pltpu.make_async_copy(t12_hbm.at[cc_v[b]], rows_v[b], gsem[b]).wait()
        pltpu.async_copy(rows_v[b], out_hbm.at[pl.ds(blkno(i) * BLK, BLK)],
                         ssem[b])

    for b in range(NBUF):
        idx_start(b, b)
    for b in range(NBUF):
        gather_start(b, b, wait_store=False)

    def group(g, carry):
        i0 = NBUF * g
        for b in range(NBUF):
            finish(i0 + b, b)
            gather_start(i0 + b + NBUF, b)
        return carry

    lax.fori_loop(0, NITER // NBUF - 2, group, 0)

    for b in range(NBUF):
        finish(NITER - 2 * NBUF + b, b)
        gather_start(NITER - NBUF + b, b, prefetch_idx=False)
    for b in range(NBUF):
        finish(NITER - NBUF + b, b)

    for b in range(NBUF):
        pltpu.make_async_copy(out_hbm.at[pl.ds(0, BLK)], rows_v[b],
                              ssem[b]).wait()


def _edge_call(z, ei, t12):
    mesh = plsc.VectorSubcoreMesh(core_axis_name="c", subcore_axis_name="s")
    n_nodes = z.shape[0]
    scratch = [pltpu.VMEM((n_nodes,), jnp.int32)]
    scratch += [pltpu.VMEM((2, WIN), jnp.int32) for _ in range(NBUF)]
    scratch += [pltpu.VMEM((BLK,), jnp.int32) for _ in range(NBUF)]
    scratch += [pltpu.VMEM((BLK, H), jnp.float32) for _ in range(NBUF)]
    scratch += [pltpu.SemaphoreType.DMA for _ in range(3 * NBUF)]
    fn = pl.kernel(
        _edge_body,
        out_type=jax.ShapeDtypeStruct((NW * NITER * BLK, H), jnp.float32),
        mesh=mesh,
        scratch_types=scratch,
        compiler_params=pltpu.CompilerParams(needs_layout_passes=False),
    )
    return fn(z, ei, t12)


def kernel(z, edge_index, emb_table, W, b):
    t12 = _build_t12(emb_table, W, b)
    out = _edge_call(z.astype(jnp.int32), edge_index.astype(jnp.int32), t12)
    return out[:, :, None, None]


# speedup vs baseline: 1.0000x; 1.0000x over previous
import jax
import jax.numpy as jnp
from jax import lax
from jax.experimental import pallas as pl
from jax.experimental.pallas import tpu as pltpu
from jax.experimental.pallas import tpu_sc as plsc

H = 128
NCLS = 128
BLK = 80
NW = 32
NITER = 125
NBUF = 5
WIN = 256


def _t12_body(emb_ref, w_ref, b_ref, out_ref):
    emb = emb_ref[...]
    t1 = jnp.dot(emb, w_ref[:H, :], preferred_element_type=jnp.float32)
    t2 = jnp.dot(emb, w_ref[H:, :], preferred_element_type=jnp.float32)
    t1 = t1 + b_ref[...]
    out_ref[...] = t1[:, None, :] + t2[None, :, :]


def _build_t12(emb_table, W, b):
    out = pl.pallas_call(
        _t12_body,
        out_shape=jax.ShapeDtypeStruct((NCLS, NCLS, H), jnp.float32),
    )(emb_table, W, b.reshape(1, H))
    return out.reshape(NCLS * NCLS, H)


def _edge_body(z_hbm, ei_hbm, t12_hbm, out_hbm, *scr):
    z_v = scr[0]
    sd_v = scr[1:1 + NBUF]
    cc_v = scr[1 + NBUF:1 + 2 * NBUF]
    rows_v = scr[1 + 2 * NBUF:1 + 3 * NBUF]
    isem = scr[1 + 3 * NBUF:1 + 4 * NBUF]
    gsem = scr[1 + 4 * NBUF:1 + 5 * NBUF]
    ssem = scr[1 + 5 * NBUF:1 + 6 * NBUF]

    cid = lax.axis_index("c")
    sid = lax.axis_index("s")
    wid = sid * 2 + cid

    def blkno(i):
        return wid + i * NW

    pltpu.sync_copy(z_hbm, z_v)

    n_edges = NW * NITER * BLK

    def win_start(i):
        st = blkno(i) * BLK
        return jnp.minimum((st // 128) * 128, n_edges - WIN)

    def idx_start(i, b):
        pltpu.async_copy(ei_hbm.at[:, pl.ds(win_start(i), WIN)], sd_v[b],
                         isem[b])

    def gather_start(i, b, prefetch_idx=True, wait_store=True):
        pltpu.make_async_copy(ei_hbm.at[:, pl.ds(0, WIN)], sd_v[b],
                              isem[b]).wait()
        off = blkno(i) * BLK - win_start(i)
        for j in range(BLK // 16):
            s = plsc.load_gather(z_v, [sd_v[b][0, pl.ds(off + j * 16, 16)]])
            d = plsc.load_gather(z_v, [sd_v[b][1, pl.ds(off + j * 16, 16)]])
            cc_v[b][pl.ds(j * 16, 16)] = s * NCLS + d
        if prefetch_idx:
            idx_start(i + NBUF, b)
        if wait_store:
            pltpu.make_async_copy(out_hbm.at[pl.ds(0, BLK)], rows_v[b],
                                  ssem[b]).wait()
        pltpu.async_copy(t12_hbm.at[cc_v[b]], rows_v[b], gsem[b])

    def finish(i, b):
        pltpu.make_async_copy(t12_hbm.at[cc_v[b]], rows_v[b], gsem[b]).wait()
        pltpu.async_copy(rows_v[b], out_hbm.at[pl.ds(blkno(i) * BLK, BLK)],
                         ssem[b])

    for b in range(NBUF):
        idx_start(b, b)
    for b in range(NBUF):
        gather_start(b, b, wait_store=False)

    def group(g, carry):
        i0 = NBUF * g
        for b in range(NBUF):
            finish(i0 + b, b)
            gather_start(i0 + b + NBUF, b)
        return carry

    lax.fori_loop(0, NITER // NBUF - 2, group, 0)

    for b in range(NBUF):
        finish(NITER - 2 * NBUF + b, b)
        gather_start(NITER - NBUF + b, b, prefetch_idx=False)
    for b in range(NBUF):
        finish(NITER - NBUF + b, b)

    for b in range(NBUF):
        pltpu.make_async_copy(out_hbm.at[pl.ds(0, BLK)], rows_v[b],
                              ssem[b]).wait()


def _edge_call(z, ei, t12):
    mesh = plsc.VectorSubcoreMesh(core_axis_name="c", subcore_axis_name="s")
    n_nodes = z.shape[0]
    scratch = [pltpu.VMEM((n_nodes,), jnp.int32)]
    scratch += [pltpu.VMEM((2, WIN), jnp.int32) for _ in range(NBUF)]
    scratch += [pltpu.VMEM((BLK,), jnp.int32) for _ in range(NBUF)]
    scratch += [pltpu.VMEM((BLK, H), jnp.float32) for _ in range(NBUF)]
    scratch += [pltpu.SemaphoreType.DMA for _ in range(3 * NBUF)]
    fn = pl.kernel(
        _edge_body,
        out_type=jax.ShapeDtypeStruct((NW * NITER * BLK, H), jnp.float32),
        mesh=mesh,
        scratch_types=scratch,
        compiler_params=pltpu.CompilerParams(needs_layout_passes=False),
    )
    return fn(z, ei, t12)


def kernel(z, edge_index, emb_table, W, b):
    t12 = _build_t12(emb_table, W, b)
    out = _edge_call(z.astype(jnp.int32), edge_index.astype(jnp.int32), t12)
    return out[:, :, None, None]
